# baseline (speedup 1.0000x reference)
import functools

import jax
import jax.numpy as jnp
from jax import lax
from jax.experimental import pallas as pl
from jax.experimental.pallas import tpu as pltpu
from jax.experimental.pallas import tpu_sc as plsc

NC = 2
NS = 16
NW = NC * NS

NSAMP = 16384
NCOL = 26
D = 128
S_PER_W = NSAMP // NW
GR = 64
KPC = S_PER_W // GR
NCHUNK = NCOL * KPC
NBUF = 8
LA = 4
NROUND = NCHUNK // NBUF


def _body(idx_hbm, table_hbm, out_hbm, idx_v, rows_v, *sems):
    sems_g, sems_w = sems[:NBUF], sems[NBUF:]
    wid = lax.axis_index("s") * NC + lax.axis_index("c")
    sample_base = wid * S_PER_W
    pltpu.sync_copy(idx_hbm.at[:, pl.ds(sample_base, S_PER_W)], idx_v)

    def idx_vec(g):
        return idx_v.at[g // KPC].at[pl.ds((g % KPC) * GR, GR)]

    def start_gather(g, b):
        pltpu.async_copy(table_hbm.at[idx_vec(g)], rows_v.at[b], sems_g[b])

    def wait_gather(b):
        pltpu.make_async_copy(
            table_hbm.at[idx_vec(0)], rows_v.at[b], sems_g[b]).wait()

    def wait_write(b):
        pltpu.make_async_copy(
            rows_v.at[b], out_hbm.at[0].at[pl.ds(0, GR)], sems_w[b]).wait()

    def visit(g, b, do_wait_w, do_gather):
        bw = (b + LA) % NBUF
        if do_wait_w:
            wait_write(bw)
        if do_gather:
            start_gather(g + LA, bw)
        wait_gather(b)
        c = g // KPC
        j0 = sample_base + (g % KPC) * GR
        pltpu.async_copy(rows_v.at[b], out_hbm.at[c].at[pl.ds(j0, GR)],
                         sems_w[b])

    for k in range(LA):
        start_gather(k, k)

    for b in range(NBUF):
        visit(b, b, b >= NBUF - LA, True)

    def round_body(r, carry):
        g0 = r * NBUF
        for b in range(NBUF):
            visit(g0 + b, b, True, True)
        return carry

    lax.fori_loop(1, NROUND - 1, round_body, 0)

    g0 = (NROUND - 1) * NBUF
    for b in range(NBUF):
        visit(g0 + b, b, True, g0 + b + LA < NCHUNK)

    for b in range(LA, NBUF):
        wait_write(b)


_gather_call = functools.partial(
    pl.kernel,
    out_type=jax.ShapeDtypeStruct((NCOL, NSAMP, D), jnp.float32),
    mesh=plsc.VectorSubcoreMesh(core_axis_name="c", subcore_axis_name="s"),
    scratch_types=[
        pltpu.VMEM((NCOL, S_PER_W), jnp.int32),
        pltpu.VMEM((NBUF, GR, D), jnp.float32),
    ] + [pltpu.SemaphoreType.DMA] * (2 * NBUF),
    compiler_params=pltpu.CompilerParams(use_tc_tiling_on_sc=True),
)(_body)


@jax.jit
def kernel(indices, embedding_table):
    idx = indices.astype(jnp.int32).T
    out = _gather_call(idx, embedding_table)
    return out.transpose(1, 0, 2)
